# baseline (speedup 1.0000x reference)
import jax
import jax.numpy as jnp
from jax.experimental import pallas as pl

N = 1024
D = 64
H = 4
_DIMN = (((1,), (1,)), ((), ()))


def _gat_layer_inside(h, W, a_s, a_d, b, ones_col):
    hw = jnp.dot(h, W, preferred_element_type=jnp.float32)
    acc = None
    for k in range(H):
        hh = hw[:, D * k:D * (k + 1)]
        hh1 = jnp.concatenate([hh, ones_col], axis=1)
        ak_s = a_s[k:k + 1, :]
        ak_d = a_d[k:k + 1, :]
        as_row = jax.lax.dot_general(ak_s, hh, _DIMN,
                                     preferred_element_type=jnp.float32)
        ad_col = jnp.sum(hh * ak_d, axis=1, keepdims=True)
        A = jnp.max(as_row)
        ea = jnp.exp(as_row - A)
        eb = jnp.exp(0.2 * (as_row - A))
        R = A - jnp.min(as_row)
        c = ad_col + A
        g = jnp.exp(0.8 * jnp.minimum(c, jnp.minimum(R, 75.0)))
        w = jnp.maximum(g * ea, eb)
        nd = jnp.dot(w.astype(jnp.bfloat16), hh1.astype(jnp.bfloat16),
                     preferred_element_type=jnp.float32)
        contrib = nd[:, :D] / nd[:, D:D + 1]
        acc = contrib if acc is None else acc + contrib
    return acc * (1.0 / H) + b


def _model_kernel(x_ref, We_ref, be_ref, W1_ref, as1_ref, ad1_ref, b1_ref,
                  W2_ref, as2_ref, ad2_ref, b2_ref, Wa_ref, ba_ref,
                  Wc_ref, bc_ref, logits_ref, values_ref):
    x = x_ref[...]
    ones_col = jnp.ones((N, 1), jnp.float32)
    h = jnp.maximum(jnp.dot(x, We_ref[...],
                            preferred_element_type=jnp.float32)
                    + be_ref[...], 0.0)
    h = jnp.maximum(_gat_layer_inside(h, W1_ref[...], as1_ref[...],
                                      ad1_ref[...], b1_ref[...], ones_col), 0.0)
    h = jnp.maximum(_gat_layer_inside(h, W2_ref[...], as2_ref[...],
                                      ad2_ref[...], b2_ref[...], ones_col), 0.0)
    lg = jnp.dot(h, Wa_ref[...], preferred_element_type=jnp.float32) \
        + ba_ref[...]
    lg = jnp.clip(lg, -5.0, 5.0)
    col = jax.lax.broadcasted_iota(jnp.int32, lg.shape, 1)
    logits_ref[...] = jnp.where(col == 1, jnp.abs(lg), lg)
    values_ref[...] = jnp.dot(h, Wc_ref[...],
                              preferred_element_type=jnp.float32) + bc_ref[...]


def kernel(x, We, be, W1, as1, ad1, b1, W2, as2, ad2, b2, Wa, ba, Wc, bc):
    logits, values = pl.pallas_call(
        _model_kernel,
        out_shape=(jax.ShapeDtypeStruct((N, 2), jnp.float32),
                   jax.ShapeDtypeStruct((N, 1), jnp.float32)),
    )(x, We, be.reshape(1, D), W1, as1, ad1, b1.reshape(1, D),
      W2, as2, ad2, b2.reshape(1, D), Wa, ba.reshape(1, 2),
      Wc, bc.reshape(1, 1))
    return (logits, values.reshape(-1))


# speedup vs baseline: 1.4079x; 1.4079x over previous
import jax
import jax.numpy as jnp
from jax.experimental import pallas as pl

N = 1024
D = 64
H = 4
_DIMN = (((1,), (1,)), ((), ()))


def _gat_layer_inside(h, W, a_s, a_d, b, ones_col):
    ws_rows = [jax.lax.dot_general(a_s[k:k + 1, :], W[:, D * k:D * (k + 1)],
                                   _DIMN, preferred_element_type=jnp.float32)
               for k in range(H)]
    wd_rows = [jax.lax.dot_general(a_d[k:k + 1, :], W[:, D * k:D * (k + 1)],
                                   _DIMN, preferred_element_type=jnp.float32)
               for k in range(H)]
    Wd = jnp.concatenate(wd_rows, axis=0)
    ad_all = jax.lax.dot_general(h, Wd, _DIMN,
                                 preferred_element_type=jnp.float32)
    as_rows, A_parts, Rc_parts = [], [], []
    for k in range(H):
        as_row = jax.lax.dot_general(ws_rows[k], h, _DIMN,
                                     preferred_element_type=jnp.float32)
        A = jnp.max(as_row)
        R = A - jnp.min(as_row)
        as_rows.append(as_row)
        A_parts.append(jnp.broadcast_to(A, (1, 1)))
        Rc_parts.append(jnp.broadcast_to(jnp.minimum(R, 75.0), (1, 1)))
    A_row = jnp.concatenate(A_parts, axis=1)
    Rc_row = jnp.concatenate(Rc_parts, axis=1)
    g_all = jnp.exp(0.8 * jnp.minimum(ad_all + A_row, Rc_row))
    hw = jnp.dot(h, W, preferred_element_type=jnp.float32)
    acc = None
    for k in range(H):
        hh1 = jnp.concatenate([hw[:, D * k:D * (k + 1)], ones_col], axis=1)
        A = A_parts[k][0, 0]
        ea = jnp.exp(as_rows[k] - A)
        eb = jnp.exp(0.2 * (as_rows[k] - A))
        w = jnp.maximum(g_all[:, k:k + 1] * ea, eb)
        nd = jnp.dot(w, hh1, preferred_element_type=jnp.float32)
        contrib = nd[:, :D] / nd[:, D:D + 1]
        acc = contrib if acc is None else acc + contrib
    return acc * (1.0 / H) + b


def _model_kernel(x_ref, We_ref, be_ref, W1_ref, as1_ref, ad1_ref, b1_ref,
                  W2_ref, as2_ref, ad2_ref, b2_ref, Wa_ref, ba_ref,
                  Wc_ref, bc_ref, logits_ref, values_ref):
    x = x_ref[...]
    ones_col = jnp.ones((N, 1), jnp.float32)
    h = jnp.maximum(jnp.dot(x, We_ref[...],
                            preferred_element_type=jnp.float32)
                    + be_ref[...], 0.0)
    h = jnp.maximum(_gat_layer_inside(h, W1_ref[...], as1_ref[...],
                                      ad1_ref[...], b1_ref[...], ones_col), 0.0)
    h = jnp.maximum(_gat_layer_inside(h, W2_ref[...], as2_ref[...],
                                      ad2_ref[...], b2_ref[...], ones_col), 0.0)
    lg = jnp.dot(h, Wa_ref[...], preferred_element_type=jnp.float32) \
        + ba_ref[...]
    lg = jnp.clip(lg, -5.0, 5.0)
    col = jax.lax.broadcasted_iota(jnp.int32, lg.shape, 1)
    logits_ref[...] = jnp.where(col == 1, jnp.abs(lg), lg)
    values_ref[...] = jnp.dot(h, Wc_ref[...],
                              preferred_element_type=jnp.float32) + bc_ref[...]


def kernel(x, We, be, W1, as1, ad1, b1, W2, as2, ad2, b2, Wa, ba, Wc, bc):
    logits, values = pl.pallas_call(
        _model_kernel,
        out_shape=(jax.ShapeDtypeStruct((N, 2), jnp.float32),
                   jax.ShapeDtypeStruct((N, 1), jnp.float32)),
    )(x, We, be.reshape(1, D), W1, as1, ad1, b1.reshape(1, D),
      W2, as2, ad2, b2.reshape(1, D), Wa, ba.reshape(1, 2),
      Wc, bc.reshape(1, 1))
    return (logits, values.reshape(-1))


# speedup vs baseline: 1.4159x; 1.0057x over previous
import jax
import jax.numpy as jnp
from jax.experimental import pallas as pl

N = 1024
D = 64
H = 4
_DIMN = (((1,), (1,)), ((), ()))


def _gat_layer_inside(h, W, a_s, a_d, b, ones_col):
    ws_rows = [jax.lax.dot_general(a_s[k:k + 1, :], W[:, D * k:D * (k + 1)],
                                   _DIMN, preferred_element_type=jnp.float32)
               for k in range(H)]
    wd_rows = [jax.lax.dot_general(a_d[k:k + 1, :], W[:, D * k:D * (k + 1)],
                                   _DIMN, preferred_element_type=jnp.float32)
               for k in range(H)]
    Wd = jnp.concatenate(wd_rows, axis=0)
    ad_all = jax.lax.dot_general(h, Wd, _DIMN,
                                 preferred_element_type=jnp.float32)
    as_rows, A_parts, Rc_parts = [], [], []
    for k in range(H):
        as_row = jax.lax.dot_general(ws_rows[k], h, _DIMN,
                                     preferred_element_type=jnp.float32)
        A = jnp.max(as_row)
        R = A - jnp.min(as_row)
        as_rows.append(as_row)
        A_parts.append(jnp.broadcast_to(A, (1, 1)))
        Rc_parts.append(jnp.broadcast_to(jnp.minimum(R, 75.0), (1, 1)))
    A_row = jnp.concatenate(A_parts, axis=1)
    Rc_row = jnp.concatenate(Rc_parts, axis=1)
    g_all = jnp.exp(0.8 * jnp.minimum(ad_all + A_row, Rc_row))
    hw = jnp.dot(h, W, preferred_element_type=jnp.float32)
    acc = None
    for k in range(H):
        hh1 = jnp.concatenate([hw[:, D * k:D * (k + 1)], ones_col], axis=1)
        A = A_parts[k][0, 0]
        ea = jnp.exp(as_rows[k] - A).astype(jnp.bfloat16)
        eb = jnp.exp(0.2 * (as_rows[k] - A)).astype(jnp.bfloat16)
        gk = g_all[:, k:k + 1].astype(jnp.bfloat16)
        w = jnp.maximum(gk * ea, eb)
        nd = jnp.dot(w, hh1.astype(jnp.bfloat16),
                     preferred_element_type=jnp.float32)
        contrib = nd[:, :D] / nd[:, D:D + 1]
        acc = contrib if acc is None else acc + contrib
    return acc * (1.0 / H) + b


def _model_kernel(x_ref, We_ref, be_ref, W1_ref, as1_ref, ad1_ref, b1_ref,
                  W2_ref, as2_ref, ad2_ref, b2_ref, Wa_ref, ba_ref,
                  Wc_ref, bc_ref, logits_ref, values_ref):
    x = x_ref[...]
    ones_col = jnp.ones((N, 1), jnp.float32)
    h = jnp.maximum(jnp.dot(x, We_ref[...],
                            preferred_element_type=jnp.float32)
                    + be_ref[...], 0.0)
    h = jnp.maximum(_gat_layer_inside(h, W1_ref[...], as1_ref[...],
                                      ad1_ref[...], b1_ref[...], ones_col), 0.0)
    h = jnp.maximum(_gat_layer_inside(h, W2_ref[...], as2_ref[...],
                                      ad2_ref[...], b2_ref[...], ones_col), 0.0)
    lg = jnp.dot(h, Wa_ref[...], preferred_element_type=jnp.float32) \
        + ba_ref[...]
    lg = jnp.clip(lg, -5.0, 5.0)
    col = jax.lax.broadcasted_iota(jnp.int32, lg.shape, 1)
    logits_ref[...] = jnp.where(col == 1, jnp.abs(lg), lg)
    values_ref[...] = jnp.dot(h, Wc_ref[...],
                              preferred_element_type=jnp.float32) + bc_ref[...]


def kernel(x, We, be, W1, as1, ad1, b1, W2, as2, ad2, b2, Wa, ba, Wc, bc):
    logits, values = pl.pallas_call(
        _model_kernel,
        out_shape=(jax.ShapeDtypeStruct((N, 2), jnp.float32),
                   jax.ShapeDtypeStruct((N, 1), jnp.float32)),
    )(x, We, be.reshape(1, D), W1, as1, ad1, b1.reshape(1, D),
      W2, as2, ad2, b2.reshape(1, D), Wa, ba.reshape(1, 2),
      Wc, bc.reshape(1, 1))
    return (logits, values.reshape(-1))


# speedup vs baseline: 1.4702x; 1.0384x over previous
import jax
import jax.numpy as jnp
from jax.experimental import pallas as pl

N = 1024
D = 64
H = 4
_DIMN = (((1,), (1,)), ((), ()))


def _gat_layer_inside(h, W, a_s, a_d, b, ones_col):
    ws_rows = [jax.lax.dot_general(a_s[k:k + 1, :], W[:, D * k:D * (k + 1)],
                                   _DIMN, preferred_element_type=jnp.float32)
               for k in range(H)]
    wd_rows = [jax.lax.dot_general(a_d[k:k + 1, :], W[:, D * k:D * (k + 1)],
                                   _DIMN, preferred_element_type=jnp.float32)
               for k in range(H)]
    Wd = jnp.concatenate(wd_rows, axis=0)
    Ws = jnp.concatenate(ws_rows, axis=0)
    ad_all = jax.lax.dot_general(h, Wd, _DIMN,
                                 preferred_element_type=jnp.float32)
    as_all = jax.lax.dot_general(Ws, h, _DIMN,
                                 preferred_element_type=jnp.float32)
    A_col = jnp.max(as_all, axis=1, keepdims=True)
    Rc_col = jnp.minimum(A_col - jnp.min(as_all, axis=1, keepdims=True), 75.0)
    as_rows = [as_all[k:k + 1, :] for k in range(H)]
    A_parts = [A_col[k:k + 1, 0:1] for k in range(H)]
    A_row = jnp.concatenate(A_parts, axis=1)
    Rc_row = jnp.concatenate([Rc_col[k:k + 1, 0:1] for k in range(H)],
                             axis=1)
    g_all = jnp.exp(0.8 * jnp.minimum(ad_all + A_row, Rc_row)
                    ).astype(jnp.bfloat16)
    hw = jnp.dot(h, W,
                 preferred_element_type=jnp.float32).astype(jnp.bfloat16)
    acc = None
    for k in range(H):
        hh1 = jnp.concatenate([hw[:, D * k:D * (k + 1)], ones_col], axis=1)
        A = A_parts[k][0, 0]
        sa = as_rows[k] - A
        ea = jnp.exp(sa).astype(jnp.bfloat16)
        eb = jnp.exp(0.2 * sa).astype(jnp.bfloat16)
        w = jnp.maximum(g_all[:, k:k + 1] * ea, eb)
        nd = jnp.dot(w, hh1, preferred_element_type=jnp.float32)
        contrib = nd[:, :D] / nd[:, D:D + 1]
        acc = contrib if acc is None else acc + contrib
    return acc * (1.0 / H) + b


def _model_kernel(x_ref, We_ref, be_ref, W1_ref, as1_ref, ad1_ref, b1_ref,
                  W2_ref, as2_ref, ad2_ref, b2_ref, Wa_ref, ba_ref,
                  Wc_ref, bc_ref, logits_ref, values_ref):
    x = x_ref[...]
    ones_col = jnp.ones((N, 1), jnp.bfloat16)
    h = jnp.maximum(jnp.dot(x, We_ref[...],
                            preferred_element_type=jnp.float32)
                    + be_ref[...], 0.0)
    h = jnp.maximum(_gat_layer_inside(h, W1_ref[...], as1_ref[...],
                                      ad1_ref[...], b1_ref[...], ones_col), 0.0)
    h = jnp.maximum(_gat_layer_inside(h, W2_ref[...], as2_ref[...],
                                      ad2_ref[...], b2_ref[...], ones_col), 0.0)
    Wo = jnp.concatenate([Wa_ref[...], Wc_ref[...]], axis=1)
    bo = jnp.concatenate([ba_ref[...], bc_ref[...]], axis=1)
    lgv = jnp.dot(h, Wo, preferred_element_type=jnp.float32) + bo
    cl = jnp.clip(lgv, -5.0, 5.0)
    col = jax.lax.broadcasted_iota(jnp.int32, lgv.shape, 1)
    res = jnp.where(col == 1, jnp.abs(cl), cl)
    logits_ref[...] = res[:, 0:2]
    values_ref[...] = lgv[:, 2:3]


def kernel(x, We, be, W1, as1, ad1, b1, W2, as2, ad2, b2, Wa, ba, Wc, bc):
    logits, values = pl.pallas_call(
        _model_kernel,
        out_shape=(jax.ShapeDtypeStruct((N, 2), jnp.float32),
                   jax.ShapeDtypeStruct((N, 1), jnp.float32)),
    )(x, We, be.reshape(1, D), W1, as1, ad1, b1.reshape(1, D),
      W2, as2, ad2, b2.reshape(1, D), Wa, ba.reshape(1, 2),
      Wc, bc.reshape(1, 1))
    return (logits, values.reshape(-1))


# speedup vs baseline: 1.5937x; 1.0840x over previous
import jax
import jax.numpy as jnp
from jax.experimental import pallas as pl

N = 1024
D = 64
H = 4
_DIMN = (((1,), (1,)), ((), ()))


def _gat_layer_inside(h, W, a_s, a_d, b, ones_col):
    ws_rows = [jax.lax.dot_general(a_s[k:k + 1, :], W[:, D * k:D * (k + 1)],
                                   _DIMN, preferred_element_type=jnp.float32)
               for k in range(H)]
    wd_rows = [jax.lax.dot_general(a_d[k:k + 1, :], W[:, D * k:D * (k + 1)],
                                   _DIMN, preferred_element_type=jnp.float32)
               for k in range(H)]
    Wd = jnp.concatenate(wd_rows, axis=0)
    Ws = jnp.concatenate(ws_rows, axis=0)
    ad_all = jax.lax.dot_general(h, Wd, _DIMN,
                                 preferred_element_type=jnp.float32)
    as_all = jax.lax.dot_general(Ws, h, _DIMN,
                                 preferred_element_type=jnp.float32)
    A_col = jnp.max(as_all, axis=1, keepdims=True)
    Rc_col = jnp.minimum(A_col - jnp.min(as_all, axis=1, keepdims=True), 75.0)
    as_rows = [as_all[k:k + 1, :] for k in range(H)]
    A_parts = [A_col[k:k + 1, 0:1] for k in range(H)]
    A_row = jnp.concatenate(A_parts, axis=1)
    Rc_row = jnp.concatenate([Rc_col[k:k + 1, 0:1] for k in range(H)],
                             axis=1)
    g_all = jnp.exp(0.8 * jnp.minimum(ad_all + A_row, Rc_row)
                    ).astype(jnp.bfloat16)
    hw = jnp.dot(h, W,
                 preferred_element_type=jnp.float32).astype(jnp.bfloat16)
    acc = None
    for k in range(H):
        hh1 = jnp.concatenate([hw[:, D * k:D * (k + 1)], ones_col], axis=1)
        A = A_parts[k][0, 0]
        sa = as_rows[k] - A
        ea = jnp.exp(sa).astype(jnp.bfloat16)
        eb = jnp.exp(0.2 * sa).astype(jnp.bfloat16)
        w = jnp.maximum(g_all[:, k:k + 1] * ea, eb)
        nd = jnp.dot(w, hh1, preferred_element_type=jnp.float32)
        contrib = nd[:, :D] / nd[:, D:D + 1]
        acc = contrib if acc is None else acc + contrib
    return acc * (1.0 / H) + b


def _model_kernel(x_ref, We_ref, be_ref, W1_ref, as1_ref, ad1_ref, b1_ref,
                  W2_ref, as2_ref, ad2_ref, b2_ref, Wa_ref, ba_ref,
                  Wc_ref, bc_ref, logits_ref, values_ref):
    x = x_ref[...]
    ones_col = jnp.ones((N, 1), jnp.bfloat16)
    h = jnp.maximum(jnp.dot(x, We_ref[...],
                            preferred_element_type=jnp.float32)
                    + be_ref[...], 0.0)
    h = jnp.maximum(_gat_layer_inside(h, W1_ref[...], as1_ref[...],
                                      ad1_ref[...], b1_ref[...], ones_col), 0.0)
    h = jnp.maximum(_gat_layer_inside(h, W2_ref[...], as2_ref[...],
                                      ad2_ref[...], b2_ref[...], ones_col), 0.0)
    Wo = jnp.concatenate([Wa_ref[...], Wc_ref[...]], axis=1)
    bo = jnp.concatenate([ba_ref[...], bc_ref[...]], axis=0)
    lgv = jnp.dot(h, Wo, preferred_element_type=jnp.float32) + bo
    cl = jnp.clip(lgv, -5.0, 5.0)
    col = jax.lax.broadcasted_iota(jnp.int32, lgv.shape, 1)
    res = jnp.where(col == 1, jnp.abs(cl), cl)
    logits_ref[...] = res[:, 0:2]
    values_ref[...] = lgv[:, 2]


def kernel(x, We, be, W1, as1, ad1, b1, W2, as2, ad2, b2, Wa, ba, Wc, bc):
    logits, values = pl.pallas_call(
        _model_kernel,
        out_shape=(jax.ShapeDtypeStruct((N, 2), jnp.float32),
                   jax.ShapeDtypeStruct((N,), jnp.float32)),
    )(x, We, be, W1, as1, ad1, b1, W2, as2, ad2, b2, Wa, ba, Wc, bc)
    return (logits, values)


# speedup vs baseline: 1.5955x; 1.0011x over previous
import jax
import jax.numpy as jnp
from jax.experimental import pallas as pl

N = 1024
D = 64
H = 4
_DIMN = (((1,), (1,)), ((), ()))


def _gat_layer_inside(h, W, a_s, a_d, b, ones_col):
    ws_rows = [jax.lax.dot_general(a_s[k:k + 1, :], W[:, D * k:D * (k + 1)],
                                   _DIMN, preferred_element_type=jnp.float32)
               for k in range(H)]
    wd_rows = [jax.lax.dot_general(a_d[k:k + 1, :], W[:, D * k:D * (k + 1)],
                                   _DIMN, preferred_element_type=jnp.float32)
               for k in range(H)]
    Wd = jnp.concatenate(wd_rows, axis=0)
    Ws = jnp.concatenate(ws_rows, axis=0)
    ad_all = jax.lax.dot_general(h, Wd, _DIMN,
                                 preferred_element_type=jnp.float32)
    as_all = jax.lax.dot_general(Ws, h, _DIMN,
                                 preferred_element_type=jnp.float32)
    A_col = jnp.max(as_all, axis=1, keepdims=True)
    Rc_col = jnp.minimum(A_col - jnp.min(as_all, axis=1, keepdims=True), 75.0)
    as_rows = [as_all[k:k + 1, :] for k in range(H)]
    A_parts = [A_col[k:k + 1, 0:1] for k in range(H)]
    A_row = jnp.concatenate(A_parts, axis=1)
    Rc_row = jnp.concatenate([Rc_col[k:k + 1, 0:1] for k in range(H)],
                             axis=1)
    g_all = jnp.exp(0.8 * jnp.minimum(ad_all + A_row, Rc_row)
                    ).astype(jnp.bfloat16)
    hw = jnp.dot(h, W,
                 preferred_element_type=jnp.float32).astype(jnp.bfloat16)
    acc = None
    for k in range(H):
        hh1 = jnp.concatenate([hw[:, D * k:D * (k + 1)], ones_col], axis=1)
        A = A_parts[k][0, 0]
        sa = as_rows[k] - A
        ea = jnp.exp(sa).astype(jnp.bfloat16)
        eb = jnp.exp(0.2 * sa).astype(jnp.bfloat16)
        w = jnp.maximum(g_all[:, k:k + 1] * ea, eb)
        nd = jnp.dot(w, hh1, preferred_element_type=jnp.float32)
        contrib = nd[:, :D] / nd[:, D:D + 1]
        acc = contrib if acc is None else acc + contrib
    return acc + b


def _model_kernel(x_ref, We_ref, be_ref, W1_ref, as1_ref, ad1_ref, b1_ref,
                  W2_ref, as2_ref, ad2_ref, b2_ref, Wa_ref, ba_ref,
                  Wc_ref, bc_ref, logits_ref, values_ref):
    x = x_ref[...]
    ones_col = jnp.full((N, 1), float(H), jnp.bfloat16)
    h = jnp.maximum(jnp.dot(x, We_ref[...],
                            preferred_element_type=jnp.float32)
                    + be_ref[...], 0.0)
    h = jnp.maximum(_gat_layer_inside(h, W1_ref[...], as1_ref[...],
                                      ad1_ref[...], b1_ref[...], ones_col), 0.0)
    h = jnp.maximum(_gat_layer_inside(h, W2_ref[...], as2_ref[...],
                                      ad2_ref[...], b2_ref[...], ones_col), 0.0)
    Wo = jnp.concatenate([Wa_ref[...], Wc_ref[...]], axis=1)
    bo = jnp.concatenate([ba_ref[...], bc_ref[...]], axis=0)
    lgv = jnp.dot(h, Wo, preferred_element_type=jnp.float32) + bo
    cl = jnp.clip(lgv, -5.0, 5.0)
    col = jax.lax.broadcasted_iota(jnp.int32, lgv.shape, 1)
    res = jnp.where(col == 1, jnp.abs(cl), cl)
    logits_ref[...] = res[:, 0:2]
    values_ref[...] = lgv[:, 2]


def kernel(x, We, be, W1, as1, ad1, b1, W2, as2, ad2, b2, Wa, ba, Wc, bc):
    logits, values = pl.pallas_call(
        _model_kernel,
        out_shape=(jax.ShapeDtypeStruct((N, 2), jnp.float32),
                   jax.ShapeDtypeStruct((N,), jnp.float32)),
    )(x, We, be, W1, as1, ad1, b1, W2, as2, ad2, b2, Wa, ba, Wc, bc)
    return (logits, values)
